# baseline (speedup 1.0000x reference)
import jax
import jax.numpy as jnp
from jax.experimental import pallas as pl
from jax.experimental.pallas import tpu as pltpu

E = 16
D = 1024
H = 4096
R = 4
BATCH = 32

HB = 1024
DB = 256
P1 = H // HB
P2 = D // DB


def _gate_kernel(x_ref, wg_ref, bg_ref, w_ref, idx_ref):
    x = x_ref[...]
    logits = jax.lax.dot_general(
        x, wg_ref[...], (((1,), (1,)), ((), ())),
        preferred_element_type=jnp.float32) + bg_ref[...]
    m = jnp.max(logits, axis=-1, keepdims=True)
    ex = jnp.exp(logits - m)
    probs = ex / jnp.sum(ex, axis=-1, keepdims=True)
    w_ref[...] = jnp.max(probs, axis=-1, keepdims=True)
    idx_ref[...] = jnp.argmax(probs, axis=-1, keepdims=True).astype(jnp.int32)


def _ffn_kernel(idx_ref, x_ref, w1_ref, b1_ref, a1_ref, bb1_ref,
                w2_ref, b2_ref, a2_ref, bb2_ref, w_ref, out_ref, h_ref):
    i = pl.program_id(0)

    @pl.when(i < P1)
    def _layer1():
        x = x_ref[...]
        t1 = jax.lax.dot_general(x, bb1_ref[0], (((1,), (1,)), ((), ())),
                                 preferred_element_type=jnp.float32)
        h = jax.lax.dot_general(x, w1_ref[0], (((1,), (1,)), ((), ())),
                                preferred_element_type=jnp.float32)
        h = h + jax.lax.dot_general(t1, a1_ref[0], (((1,), (1,)), ((), ())),
                                    preferred_element_type=jnp.float32)
        h_ref[:, pl.ds(i * HB, HB)] = jnp.maximum(h + b1_ref[0], 0.0)

    @pl.when(i >= P1)
    def _layer2():
        h = h_ref[...]
        p = jax.lax.dot_general(h, w2_ref[0], (((1,), (1,)), ((), ())),
                                preferred_element_type=jnp.float32)
        t2 = jax.lax.dot_general(h, bb2_ref[0], (((1,), (1,)), ((), ())),
                                 preferred_element_type=jnp.float32)
        p = p + jax.lax.dot_general(t2, a2_ref[0], (((1,), (1,)), ((), ())),
                                    preferred_element_type=jnp.float32)
        out_ref[...] = (p + b2_ref[0]) * w_ref[...]


@jax.jit
def kernel(x, Wg, bg, W1, b1, A1, B1, W2, b2, A2, B2):
    topw, topi = pl.pallas_call(
        _gate_kernel,
        out_shape=(
            jax.ShapeDtypeStruct((BATCH, 1), jnp.float32),
            jax.ShapeDtypeStruct((BATCH, 1), jnp.int32),
        ),
    )(x, Wg, bg.reshape(1, E))

    e_idx = topi[0]

    def p1(i):
        return jnp.minimum(i, P1 - 1)

    def p2(i):
        return jnp.maximum(i - P1, 0)

    grid_spec = pltpu.PrefetchScalarGridSpec(
        num_scalar_prefetch=1,
        grid=(P1 + P2,),
        in_specs=[
            pl.BlockSpec((BATCH, D), lambda i, e: (0, 0)),
            pl.BlockSpec((1, HB, D), lambda i, e: (e[0], p1(i), 0)),
            pl.BlockSpec((1, 1, HB), lambda i, e: (e[0], 0, p1(i))),
            pl.BlockSpec((1, HB, R), lambda i, e: (e[0], p1(i), 0)),
            pl.BlockSpec((1, R, D), lambda i, e: (e[0], 0, 0)),
            pl.BlockSpec((1, DB, H), lambda i, e: (e[0], p2(i), 0)),
            pl.BlockSpec((1, 1, DB), lambda i, e: (e[0], 0, p2(i))),
            pl.BlockSpec((1, DB, R), lambda i, e: (e[0], p2(i), 0)),
            pl.BlockSpec((1, R, H), lambda i, e: (e[0], 0, 0)),
            pl.BlockSpec((BATCH, 1), lambda i, e: (0, 0)),
        ],
        out_specs=pl.BlockSpec((BATCH, DB), lambda i, e: (0, p2(i))),
        scratch_shapes=[pltpu.VMEM((BATCH, H), jnp.float32)],
    )
    out = pl.pallas_call(
        _ffn_kernel,
        grid_spec=grid_spec,
        out_shape=jax.ShapeDtypeStruct((BATCH, D), jnp.float32),
    )(e_idx, x, W1, b1.reshape(E, 1, H), A1, B1, W2,
      b2.reshape(E, 1, D), A2, B2, topw)
    return (out, topi)


# speedup vs baseline: 1.0066x; 1.0066x over previous
import jax
import jax.numpy as jnp
from jax.experimental import pallas as pl
from jax.experimental.pallas import tpu as pltpu

E = 16
D = 1024
H = 4096
R = 4
BATCH = 32

NSPLIT = 4
HB = 1024
DB = 256
HBq = HB // NSPLIT
DBq = DB // NSPLIT
P1 = H // HB
P2 = D // DB


def _gate_kernel(x_ref, wg_ref, bg_ref, w_ref, idx_ref):
    x = x_ref[...]
    logits = jax.lax.dot_general(
        x, wg_ref[...], (((1,), (1,)), ((), ())),
        preferred_element_type=jnp.float32) + bg_ref[...]
    m = jnp.max(logits, axis=-1, keepdims=True)
    ex = jnp.exp(logits - m)
    probs = ex / jnp.sum(ex, axis=-1, keepdims=True)
    w_ref[...] = jnp.max(probs, axis=-1, keepdims=True)
    idx_ref[...] = jnp.argmax(probs, axis=-1, keepdims=True).astype(jnp.int32)


def _dot_nt(a, b):
    return jax.lax.dot_general(a, b, (((1,), (1,)), ((), ())),
                               preferred_element_type=jnp.float32)


def _ffn_kernel(idx_ref, x_ref, *refs):
    w1_refs = refs[0:NSPLIT]
    b1_ref, a1_ref, bb1_ref = refs[NSPLIT:NSPLIT + 3]
    w2_refs = refs[NSPLIT + 3:2 * NSPLIT + 3]
    b2_ref, a2_ref, bb2_ref, w_ref = refs[2 * NSPLIT + 3:2 * NSPLIT + 7]
    out_ref, h_ref = refs[2 * NSPLIT + 7:]
    i = pl.program_id(0)

    @pl.when(i < P1)
    def _layer1():
        x = x_ref[...]
        t1 = _dot_nt(x, bb1_ref[0])
        for q in range(NSPLIT):
            h = _dot_nt(x, w1_refs[q][0])
            h = h + _dot_nt(t1, a1_ref[0, q * HBq:(q + 1) * HBq, :])
            h = h + b1_ref[0, :, q * HBq:(q + 1) * HBq]
            h_ref[:, pl.ds(i * HB + q * HBq, HBq)] = jnp.maximum(h, 0.0)

    @pl.when(i >= P1)
    def _layer2():
        h = h_ref[...]
        t2 = _dot_nt(h, bb2_ref[0])
        for q in range(NSPLIT):
            p = _dot_nt(h, w2_refs[q][0])
            p = p + _dot_nt(t2, a2_ref[0, q * DBq:(q + 1) * DBq, :])
            p = p + b2_ref[0, :, q * DBq:(q + 1) * DBq]
            out_ref[:, q * DBq:(q + 1) * DBq] = p * w_ref[...]


@jax.jit
def kernel(x, Wg, bg, W1, b1, A1, B1, W2, b2, A2, B2):
    topw, topi = pl.pallas_call(
        _gate_kernel,
        out_shape=(
            jax.ShapeDtypeStruct((BATCH, 1), jnp.float32),
            jax.ShapeDtypeStruct((BATCH, 1), jnp.int32),
        ),
    )(x, Wg, bg.reshape(1, E))

    e_idx = topi[0]

    def p1(i):
        return jnp.minimum(i, P1 - 1)

    def p2(i):
        return jnp.maximum(i - P1, 0)

    def w1_spec(q):
        return pl.BlockSpec((1, HBq, D),
                            lambda i, e: (e[0], p1(i) * NSPLIT + q, 0))

    def w2_spec(q):
        return pl.BlockSpec((1, DBq, H),
                            lambda i, e: (e[0], p2(i) * NSPLIT + q, 0))

    grid_spec = pltpu.PrefetchScalarGridSpec(
        num_scalar_prefetch=1,
        grid=(P1 + P2,),
        in_specs=[
            pl.BlockSpec((BATCH, D), lambda i, e: (0, 0)),
            *[w1_spec(q) for q in range(NSPLIT)],
            pl.BlockSpec((1, 1, HB), lambda i, e: (e[0], 0, p1(i))),
            pl.BlockSpec((1, HB, R), lambda i, e: (e[0], p1(i), 0)),
            pl.BlockSpec((1, R, D), lambda i, e: (e[0], 0, 0)),
            *[w2_spec(q) for q in range(NSPLIT)],
            pl.BlockSpec((1, 1, DB), lambda i, e: (e[0], 0, p2(i))),
            pl.BlockSpec((1, DB, R), lambda i, e: (e[0], p2(i), 0)),
            pl.BlockSpec((1, R, H), lambda i, e: (e[0], 0, 0)),
            pl.BlockSpec((BATCH, 1), lambda i, e: (0, 0)),
        ],
        out_specs=pl.BlockSpec((BATCH, DB), lambda i, e: (0, p2(i))),
        scratch_shapes=[pltpu.VMEM((BATCH, H), jnp.float32)],
    )
    out = pl.pallas_call(
        _ffn_kernel,
        grid_spec=grid_spec,
        out_shape=jax.ShapeDtypeStruct((BATCH, D), jnp.float32),
    )(e_idx, x, W1, W1, W1, W1, b1.reshape(E, 1, H), A1, B1,
      W2, W2, W2, W2, b2.reshape(E, 1, D), A2, B2, topw)
    return (out, topi)


# speedup vs baseline: 1.0125x; 1.0058x over previous
import jax
import jax.numpy as jnp
from jax.experimental import pallas as pl
from jax.experimental.pallas import tpu as pltpu

E = 16
D = 1024
H = 4096
R = 4
BATCH = 32

NC1 = 8
NC2 = 8
CH1 = H // NC1
CH2 = D // NC2


def _gate_kernel(x_ref, wg_ref, bg_ref, w_ref, idx_ref):
    x = x_ref[...]
    logits = jax.lax.dot_general(
        x, wg_ref[...], (((1,), (1,)), ((), ())),
        preferred_element_type=jnp.float32) + bg_ref[...]
    m = jnp.max(logits, axis=-1, keepdims=True)
    ex = jnp.exp(logits - m)
    probs = ex / jnp.sum(ex, axis=-1, keepdims=True)
    w_ref[...] = jnp.max(probs, axis=-1, keepdims=True)
    idx_ref[...] = jnp.argmax(probs, axis=-1, keepdims=True).astype(jnp.int32)


def _dot_nt(a, b):
    return jax.lax.dot_general(a, b, (((1,), (1,)), ((), ())),
                               preferred_element_type=jnp.float32)


def _ffn_kernel(idx_ref, x_ref, w1_hbm, b1_ref, a1_ref, bb1_ref,
                w2_hbm, b2_ref, a2_ref, bb2_ref, w_ref, out_ref,
                w1v, w2v, h_ref, sem1, sem2):
    e = idx_ref[0]
    cps1 = [
        pltpu.make_async_copy(
            w1_hbm.at[e, pl.ds(c * CH1, CH1), :],
            w1v.at[pl.ds(c * CH1, CH1), :], sem1.at[c])
        for c in range(NC1)
    ]
    cps2 = [
        pltpu.make_async_copy(
            w2_hbm.at[e, pl.ds(c * CH2, CH2), :],
            w2v.at[pl.ds(c * CH2, CH2), :], sem2.at[c])
        for c in range(NC2)
    ]
    for cp in cps1:
        cp.start()
    for cp in cps2:
        cp.start()

    x = x_ref[...]
    t1 = _dot_nt(x, bb1_ref[0])
    for c in range(NC1):
        cps1[c].wait()
        h = _dot_nt(x, w1v[c * CH1:(c + 1) * CH1, :])
        h = h + _dot_nt(t1, a1_ref[0, c * CH1:(c + 1) * CH1, :])
        h = h + b1_ref[0, :, c * CH1:(c + 1) * CH1]
        h_ref[:, c * CH1:(c + 1) * CH1] = jnp.maximum(h, 0.0)

    hfull = h_ref[...]
    t2 = _dot_nt(hfull, bb2_ref[0])
    for c in range(NC2):
        cps2[c].wait()
        p = _dot_nt(hfull, w2v[c * CH2:(c + 1) * CH2, :])
        p = p + _dot_nt(t2, a2_ref[0, c * CH2:(c + 1) * CH2, :])
        p = p + b2_ref[0, :, c * CH2:(c + 1) * CH2]
        out_ref[:, c * CH2:(c + 1) * CH2] = p * w_ref[...]


@jax.jit
def kernel(x, Wg, bg, W1, b1, A1, B1, W2, b2, A2, B2):
    topw, topi = pl.pallas_call(
        _gate_kernel,
        out_shape=(
            jax.ShapeDtypeStruct((BATCH, 1), jnp.float32),
            jax.ShapeDtypeStruct((BATCH, 1), jnp.int32),
        ),
    )(x, Wg, bg.reshape(1, E))

    e_idx = topi[0]

    grid_spec = pltpu.PrefetchScalarGridSpec(
        num_scalar_prefetch=1,
        grid=(1,),
        in_specs=[
            pl.BlockSpec((BATCH, D), lambda i, e: (0, 0)),
            pl.BlockSpec(memory_space=pltpu.MemorySpace.HBM),
            pl.BlockSpec((1, 1, H), lambda i, e: (e[0], 0, 0)),
            pl.BlockSpec((1, H, R), lambda i, e: (e[0], 0, 0)),
            pl.BlockSpec((1, R, D), lambda i, e: (e[0], 0, 0)),
            pl.BlockSpec(memory_space=pltpu.MemorySpace.HBM),
            pl.BlockSpec((1, 1, D), lambda i, e: (e[0], 0, 0)),
            pl.BlockSpec((1, D, R), lambda i, e: (e[0], 0, 0)),
            pl.BlockSpec((1, R, H), lambda i, e: (e[0], 0, 0)),
            pl.BlockSpec((BATCH, 1), lambda i, e: (0, 0)),
        ],
        out_specs=pl.BlockSpec((BATCH, D), lambda i, e: (0, 0)),
        scratch_shapes=[
            pltpu.VMEM((H, D), jnp.float32),
            pltpu.VMEM((D, H), jnp.float32),
            pltpu.VMEM((BATCH, H), jnp.float32),
            pltpu.SemaphoreType.DMA((NC1,)),
            pltpu.SemaphoreType.DMA((NC2,)),
        ],
    )
    out = pl.pallas_call(
        _ffn_kernel,
        grid_spec=grid_spec,
        out_shape=jax.ShapeDtypeStruct((BATCH, D), jnp.float32),
    )(e_idx, x, W1, b1.reshape(E, 1, H), A1, B1, W2,
      b2.reshape(E, 1, D), A2, B2, topw)
    return (out, topi)


# speedup vs baseline: 1.9401x; 1.9162x over previous
import jax
import jax.numpy as jnp
from jax.experimental import pallas as pl
from jax.experimental.pallas import tpu as pltpu

E = 16
D = 1024
H = 4096
R = 4
BATCH = 32

NC1 = 8
NC2 = 8
CH1 = H // NC1
CH2 = D // NC2


def _gate_kernel(x_ref, wg_ref, bg_ref, w_ref, idx_ref):
    x = x_ref[...]
    logits = jax.lax.dot_general(
        x, wg_ref[...], (((1,), (1,)), ((), ())),
        preferred_element_type=jnp.float32) + bg_ref[...]
    m = jnp.max(logits, axis=-1, keepdims=True)
    ex = jnp.exp(logits - m)
    probs = ex / jnp.sum(ex, axis=-1, keepdims=True)
    w_ref[...] = jnp.max(probs, axis=-1, keepdims=True)
    idx_ref[...] = jnp.argmax(probs, axis=-1, keepdims=True).astype(jnp.int32)


def _dot_nt(a, b):
    return jax.lax.dot_general(a, b, (((1,), (1,)), ((), ())),
                               preferred_element_type=jnp.float32)


def _dot_nn(a, b):
    return jax.lax.dot_general(a, b, (((1,), (0,)), ((), ())),
                               preferred_element_type=jnp.float32)


def _ffn_kernel(idx_ref, x_ref, w1_hbm, b1_ref, a1_ref, bb1_ref,
                w2_hbm, b2_ref, a2_ref, bb2_ref, w_ref, out_ref,
                w1v, w2v, h_ref, sem1, sem2):
    e = idx_ref[0]
    cps1 = [
        pltpu.make_async_copy(
            w1_hbm.at[e, pl.ds(c * CH1, CH1), :],
            w1v.at[pl.ds(c * CH1, CH1), :], sem1.at[c])
        for c in range(NC1)
    ]
    cps2 = [
        pltpu.make_async_copy(
            w2_hbm.at[e, pl.ds(c * CH2, CH2), :],
            w2v.at[pl.ds(c * CH2, CH2), :], sem2.at[c])
        for c in range(NC2)
    ]
    for cp in cps1:
        cp.start()
    for cp in cps2:
        cp.start()

    x = x_ref[...]
    t1 = _dot_nt(x, bb1_ref[0])
    for c in range(NC1):
        cps1[c].wait()
        h = _dot_nt(x, w1v[c * CH1:(c + 1) * CH1, :])
        h = h + _dot_nn(t1, a1_ref[0, :, c * CH1:(c + 1) * CH1])
        h = h + b1_ref[0, :, c * CH1:(c + 1) * CH1]
        h_ref[:, c * CH1:(c + 1) * CH1] = jnp.maximum(h, 0.0)

    hfull = h_ref[...]
    t2 = _dot_nt(hfull, bb2_ref[0])
    for c in range(NC2):
        cps2[c].wait()
        p = _dot_nt(hfull, w2v[c * CH2:(c + 1) * CH2, :])
        p = p + _dot_nn(t2, a2_ref[0, :, c * CH2:(c + 1) * CH2])
        p = p + b2_ref[0, :, c * CH2:(c + 1) * CH2]
        out_ref[:, c * CH2:(c + 1) * CH2] = p * w_ref[...]


@jax.jit
def kernel(x, Wg, bg, W1, b1, A1, B1, W2, b2, A2, B2):
    topw, topi = pl.pallas_call(
        _gate_kernel,
        out_shape=(
            jax.ShapeDtypeStruct((BATCH, 1), jnp.float32),
            jax.ShapeDtypeStruct((BATCH, 1), jnp.int32),
        ),
    )(x, Wg, bg.reshape(1, E))

    e_idx = topi[0]

    grid_spec = pltpu.PrefetchScalarGridSpec(
        num_scalar_prefetch=1,
        grid=(1,),
        in_specs=[
            pl.BlockSpec((BATCH, D), lambda i, e: (0, 0)),
            pl.BlockSpec(memory_space=pltpu.MemorySpace.HBM),
            pl.BlockSpec((1, 1, H), lambda i, e: (e[0], 0, 0)),
            pl.BlockSpec((1, R, H), lambda i, e: (e[0], 0, 0)),
            pl.BlockSpec((1, R, D), lambda i, e: (e[0], 0, 0)),
            pl.BlockSpec(memory_space=pltpu.MemorySpace.HBM),
            pl.BlockSpec((1, 1, D), lambda i, e: (e[0], 0, 0)),
            pl.BlockSpec((1, R, D), lambda i, e: (e[0], 0, 0)),
            pl.BlockSpec((1, R, H), lambda i, e: (e[0], 0, 0)),
            pl.BlockSpec((BATCH, 1), lambda i, e: (0, 0)),
        ],
        out_specs=pl.BlockSpec((BATCH, D), lambda i, e: (0, 0)),
        scratch_shapes=[
            pltpu.VMEM((H, D), jnp.float32),
            pltpu.VMEM((D, H), jnp.float32),
            pltpu.VMEM((BATCH, H), jnp.float32),
            pltpu.SemaphoreType.DMA((NC1,)),
            pltpu.SemaphoreType.DMA((NC2,)),
        ],
    )
    out = pl.pallas_call(
        _ffn_kernel,
        grid_spec=grid_spec,
        out_shape=jax.ShapeDtypeStruct((BATCH, D), jnp.float32),
    )(e_idx, x, W1, b1.reshape(E, 1, H), A1.transpose(0, 2, 1), B1, W2,
      b2.reshape(E, 1, D), A2.transpose(0, 2, 1), B2, topw)
    return (out, topi)


# speedup vs baseline: 2.0703x; 1.0671x over previous
import jax
import jax.numpy as jnp
from jax.experimental import pallas as pl
from jax.experimental.pallas import tpu as pltpu

E = 16
D = 1024
H = 4096
R = 4
BATCH = 32

NC1 = 8
NC2 = 8
CH1 = H // NC1
CH2 = D // NC2


def _gate_kernel(x_ref, wg_ref, bg_ref, w_ref, idx_ref):
    x = x_ref[...]
    logits = jax.lax.dot_general(
        x, wg_ref[...], (((1,), (1,)), ((), ())),
        preferred_element_type=jnp.float32) + bg_ref[...]
    m = jnp.max(logits, axis=-1, keepdims=True)
    ex = jnp.exp(logits - m)
    probs = ex / jnp.sum(ex, axis=-1, keepdims=True)
    w_ref[...] = jnp.max(probs, axis=-1, keepdims=True)
    idx_ref[...] = jnp.argmax(probs, axis=-1, keepdims=True).astype(jnp.int32)


def _dot_nt(a, b):
    return jax.lax.dot_general(a, b, (((1,), (1,)), ((), ())),
                               preferred_element_type=jnp.float32)


def _dot_nn(a, b):
    return jax.lax.dot_general(a, b, (((1,), (0,)), ((), ())),
                               preferred_element_type=jnp.float32)


def _ffn_kernel(idx_ref, x_ref, w1_hbm, b1_ref, a1_ref, bb1_ref,
                w2_hbm, b2_ref, a2_ref, bb2_ref, w_ref, out_ref,
                w1v, w2v, h_ref, sem1, sem2):
    e = idx_ref[0, 0]
    cps1 = [
        pltpu.make_async_copy(
            w1_hbm.at[e, pl.ds(c * CH1, CH1), :],
            w1v.at[pl.ds(c * CH1, CH1), :], sem1.at[c])
        for c in range(NC1)
    ]
    cps2 = [
        pltpu.make_async_copy(
            w2_hbm.at[e, pl.ds(c * CH2, CH2), :],
            w2v.at[pl.ds(c * CH2, CH2), :], sem2.at[c])
        for c in range(NC2)
    ]
    for cp in cps1:
        cp.start()
    for cp in cps2:
        cp.start()

    x = x_ref[...]
    t1 = _dot_nt(x, bb1_ref[0])
    for c in range(NC1):
        cps1[c].wait()
        h = _dot_nt(x, w1v[c * CH1:(c + 1) * CH1, :])
        h = h + _dot_nn(t1, a1_ref[0, :, c * CH1:(c + 1) * CH1])
        h = h + b1_ref[0, :, c * CH1:(c + 1) * CH1]
        h_ref[:, c * CH1:(c + 1) * CH1] = jnp.maximum(h, 0.0)

    hfull = h_ref[...]
    t2 = _dot_nt(hfull, bb2_ref[0])
    for c in range(NC2):
        cps2[c].wait()
        p = _dot_nt(hfull, w2v[c * CH2:(c + 1) * CH2, :])
        p = p + _dot_nn(t2, a2_ref[0, :, c * CH2:(c + 1) * CH2])
        p = p + b2_ref[0, :, c * CH2:(c + 1) * CH2]
        out_ref[:, c * CH2:(c + 1) * CH2] = p * w_ref[...]


@jax.jit
def kernel(x, Wg, bg, W1, b1, A1, B1, W2, b2, A2, B2):
    topw, topi = pl.pallas_call(
        _gate_kernel,
        out_shape=(
            jax.ShapeDtypeStruct((BATCH, 1), jnp.float32),
            jax.ShapeDtypeStruct((BATCH, 1), jnp.int32),
        ),
    )(x, Wg, bg.reshape(1, E))

    grid_spec = pltpu.PrefetchScalarGridSpec(
        num_scalar_prefetch=1,
        grid=(1,),
        in_specs=[
            pl.BlockSpec((BATCH, D), lambda i, e: (0, 0)),
            pl.BlockSpec(memory_space=pltpu.MemorySpace.HBM),
            pl.BlockSpec((1, 1, H), lambda i, e: (e[0, 0], 0, 0)),
            pl.BlockSpec((1, R, H), lambda i, e: (e[0, 0], 0, 0)),
            pl.BlockSpec((1, R, D), lambda i, e: (e[0, 0], 0, 0)),
            pl.BlockSpec(memory_space=pltpu.MemorySpace.HBM),
            pl.BlockSpec((1, 1, D), lambda i, e: (e[0, 0], 0, 0)),
            pl.BlockSpec((1, R, D), lambda i, e: (e[0, 0], 0, 0)),
            pl.BlockSpec((1, R, H), lambda i, e: (e[0, 0], 0, 0)),
            pl.BlockSpec((BATCH, 1), lambda i, e: (0, 0)),
        ],
        out_specs=pl.BlockSpec((BATCH, D), lambda i, e: (0, 0)),
        scratch_shapes=[
            pltpu.VMEM((H, D), jnp.float32),
            pltpu.VMEM((D, H), jnp.float32),
            pltpu.VMEM((BATCH, H), jnp.float32),
            pltpu.SemaphoreType.DMA((NC1,)),
            pltpu.SemaphoreType.DMA((NC2,)),
        ],
    )
    out = pl.pallas_call(
        _ffn_kernel,
        grid_spec=grid_spec,
        out_shape=jax.ShapeDtypeStruct((BATCH, D), jnp.float32),
    )(topi, x, W1, b1.reshape(E, 1, H), A1.transpose(0, 2, 1), B1, W2,
      b2.reshape(E, 1, D), A2.transpose(0, 2, 1), B2, topw)
    return (out, topi)
